# baseline (speedup 1.0000x reference)
import functools

import jax
import jax.numpy as jnp
from jax.experimental import pallas as pl
from jax.experimental.pallas import tpu as pltpu


def _make_body(tm, n, depth):
    ni = n // tm

    def body(x_ref, a0_hbm, a1_hbm, ws_ref, w0_ref, w1_ref, b_ref, o_ref,
             y0_s, y1_s, buf0, buf1, sem0, sem1):
        x = x_ref[...]
        y0_s[...] = jnp.dot(x, w0_ref[...], preferred_element_type=jnp.float32)
        y1_s[...] = jnp.dot(x, w1_ref[...], preferred_element_type=jnp.float32)
        o_ref[...] = jnp.dot(x, ws_ref[...],
                             preferred_element_type=jnp.float32) + b_ref[...]

        def copy(hbm, buf, sem, p, slot):
            return pltpu.make_async_copy(
                hbm.at[pl.ds(p * tm, tm), :], buf.at[slot], sem.at[slot])

        for p in range(depth):
            copy(a0_hbm, buf0, sem0, p, p).start()
            copy(a1_hbm, buf1, sem1, p, p).start()

        def step(i, carry):
            slot = jax.lax.rem(i, depth)
            copy(a0_hbm, buf0, sem0, i, slot).wait()
            copy(a1_hbm, buf1, sem1, i, slot).wait()
            acc = o_ref[pl.ds(i * tm, tm), :]
            acc += jnp.dot(buf0[slot], y0_s[...],
                           preferred_element_type=jnp.float32)
            acc += jnp.dot(buf1[slot], y1_s[...],
                           preferred_element_type=jnp.float32)
            o_ref[pl.ds(i * tm, tm), :] = jnp.maximum(acc, 0.0)

            @pl.when(i + depth < ni)
            def _refill():
                copy(a0_hbm, buf0, sem0, i + depth, slot).start()
                copy(a1_hbm, buf1, sem1, i + depth, slot).start()

            return carry

        jax.lax.fori_loop(0, ni, step, 0)

    return body


@functools.partial(jax.jit, static_argnames=("tm", "depth"))
def _rgcn_manual(x, a0, a1, ws, w0, w1, b, tm=128, depth=4):
    n, f = x.shape
    u = ws.shape[1]
    b2 = b.reshape(1, u)

    out = pl.pallas_call(
        _make_body(tm, n, depth),
        grid=(1,),
        in_specs=[
            pl.BlockSpec((n, f), lambda i: (0, 0)),
            pl.BlockSpec(memory_space=pl.ANY),
            pl.BlockSpec(memory_space=pl.ANY),
            pl.BlockSpec((f, u), lambda i: (0, 0)),
            pl.BlockSpec((f, u), lambda i: (0, 0)),
            pl.BlockSpec((f, u), lambda i: (0, 0)),
            pl.BlockSpec((1, u), lambda i: (0, 0)),
        ],
        out_specs=pl.BlockSpec((n, u), lambda i: (0, 0)),
        out_shape=jax.ShapeDtypeStruct((n, u), jnp.float32),
        scratch_shapes=[
            pltpu.VMEM((n, u), jnp.float32),
            pltpu.VMEM((n, u), jnp.float32),
            pltpu.VMEM((depth, tm, n), jnp.float32),
            pltpu.VMEM((depth, tm, n), jnp.float32),
            pltpu.SemaphoreType.DMA((depth,)),
            pltpu.SemaphoreType.DMA((depth,)),
        ],
    )(x, a0, a1, ws, w0, w1, b2)
    return out


def kernel(features, A_0, A_1, self_kernel, rel_kernel_0, rel_kernel_1, bias):
    x = features[0]
    out = _rgcn_manual(x, A_0, A_1, self_kernel, rel_kernel_0, rel_kernel_1, bias)
    return out[None, ...]


# speedup vs baseline: 1.0014x; 1.0014x over previous
import functools

import jax
import jax.numpy as jnp
from jax.experimental import pallas as pl
from jax.experimental.pallas import tpu as pltpu


def _make_body(tm):
    def body(x_ref, a0_ref, a1_ref, ws_ref, w0_ref, w1_ref, b_ref, o_ref,
             y0_s, y1_s, s_s):
        i = pl.program_id(0)

        @pl.when(i == 0)
        def _prologue():
            x = x_ref[...]
            y0_s[...] = jnp.dot(x, w0_ref[...],
                                preferred_element_type=jnp.float32).astype(jnp.bfloat16)
            y1_s[...] = jnp.dot(x, w1_ref[...],
                                preferred_element_type=jnp.float32).astype(jnp.bfloat16)
            s_s[...] = jnp.dot(x, ws_ref[...],
                               preferred_element_type=jnp.float32) + b_ref[...]

        acc = s_s[pl.ds(i * tm, tm), :]
        acc += jnp.dot(a0_ref[...].astype(jnp.bfloat16), y0_s[...],
                       preferred_element_type=jnp.float32)
        acc += jnp.dot(a1_ref[...].astype(jnp.bfloat16), y1_s[...],
                       preferred_element_type=jnp.float32)
        o_ref[...] = jnp.maximum(acc, 0.0)

    return body


@functools.partial(jax.jit, static_argnames=("tm",))
def _rgcn(x, a0, a1, ws, w0, w1, b, tm=256):
    n, f = x.shape
    u = ws.shape[1]
    ni = n // tm
    b2 = b.reshape(1, u)

    out = pl.pallas_call(
        _make_body(tm),
        grid=(ni,),
        in_specs=[
            pl.BlockSpec((n, f), lambda i: (0, 0)),
            pl.BlockSpec((tm, n), lambda i: (i, 0)),
            pl.BlockSpec((tm, n), lambda i: (i, 0)),
            pl.BlockSpec((f, u), lambda i: (0, 0)),
            pl.BlockSpec((f, u), lambda i: (0, 0)),
            pl.BlockSpec((f, u), lambda i: (0, 0)),
            pl.BlockSpec((1, u), lambda i: (0, 0)),
        ],
        out_specs=pl.BlockSpec((tm, u), lambda i: (i, 0)),
        out_shape=jax.ShapeDtypeStruct((n, u), jnp.float32),
        scratch_shapes=[
            pltpu.VMEM((n, u), jnp.bfloat16),
            pltpu.VMEM((n, u), jnp.bfloat16),
            pltpu.VMEM((n, u), jnp.float32),
        ],
        compiler_params=pltpu.CompilerParams(
            dimension_semantics=("arbitrary",)),
    )(x, a0, a1, ws, w0, w1, b2)
    return out


def kernel(features, A_0, A_1, self_kernel, rel_kernel_0, rel_kernel_1, bias):
    x = features[0]
    out = _rgcn(x, A_0, A_1, self_kernel, rel_kernel_0, rel_kernel_1, bias)
    return out[None, ...]


# speedup vs baseline: 1.0058x; 1.0044x over previous
import functools

import jax
import jax.numpy as jnp
from jax.experimental import pallas as pl
from jax.experimental.pallas import tpu as pltpu


def _rgcn_body(x_k_ref, x_i_ref, a0_ref, a1_ref, ws_ref, w0_ref, w1_ref,
               b_ref, o_ref, y0_s, y1_s, acc_ref):
    i = pl.program_id(0)
    k = pl.program_id(1)
    nk = pl.num_programs(1)

    @pl.when(i == 0)
    def _compute_y():
        xk = x_k_ref[...]
        y0_s[k] = jnp.dot(xk, w0_ref[...], preferred_element_type=jnp.float32)
        y1_s[k] = jnp.dot(xk, w1_ref[...], preferred_element_type=jnp.float32)

    @pl.when(k == 0)
    def _init_acc():
        acc_ref[...] = jnp.dot(x_i_ref[...], ws_ref[...],
                               preferred_element_type=jnp.float32) + b_ref[...]

    acc_ref[...] += (
        jnp.dot(a0_ref[...], y0_s[k], preferred_element_type=jnp.float32)
        + jnp.dot(a1_ref[...], y1_s[k], preferred_element_type=jnp.float32))

    @pl.when(k == nk - 1)
    def _epilogue():
        o_ref[...] = jnp.maximum(acc_ref[...], 0.0)


@functools.partial(jax.jit, static_argnames=("tm", "tk"))
def _rgcn(x, a0, a1, ws, w0, w1, b, tm=256, tk=8192):
    n, f = x.shape
    u = ws.shape[1]
    ni = n // tm
    nk = n // tk
    b2 = b.reshape(1, u)
    out = pl.pallas_call(
        _rgcn_body,
        grid=(ni, nk),
        in_specs=[
            pl.BlockSpec((tk, f), lambda i, k: (jnp.where(i == 0, k, 0), 0)),
            pl.BlockSpec((tm, f), lambda i, k: (i, 0)),
            pl.BlockSpec((tm, tk), lambda i, k: (i, k)),
            pl.BlockSpec((tm, tk), lambda i, k: (i, k)),
            pl.BlockSpec((f, u), lambda i, k: (0, 0)),
            pl.BlockSpec((f, u), lambda i, k: (0, 0)),
            pl.BlockSpec((f, u), lambda i, k: (0, 0)),
            pl.BlockSpec((1, u), lambda i, k: (0, 0)),
        ],
        out_specs=pl.BlockSpec((tm, u), lambda i, k: (i, 0)),
        out_shape=jax.ShapeDtypeStruct((n, u), jnp.float32),
        scratch_shapes=[
            pltpu.VMEM((nk, tk, u), jnp.float32),
            pltpu.VMEM((nk, tk, u), jnp.float32),
            pltpu.VMEM((tm, u), jnp.float32),
        ],
        compiler_params=pltpu.CompilerParams(
            dimension_semantics=("arbitrary", "arbitrary")),
    )(x, x, a0, a1, ws, w0, w1, b2)
    return out


def kernel(features, A_0, A_1, self_kernel, rel_kernel_0, rel_kernel_1, bias):
    x = features[0]
    out = _rgcn(x, A_0, A_1, self_kernel, rel_kernel_0, rel_kernel_1, bias)
    return out[None, ...]


# speedup vs baseline: 1.0072x; 1.0014x over previous
import functools

import jax
import jax.numpy as jnp
from jax.experimental import pallas as pl
from jax.experimental.pallas import tpu as pltpu


def _rgcn_body(x_k_ref, x_i_ref, a0_ref, a0b_ref, a1_ref, a1b_ref, ws_ref,
               w0_ref, w1_ref, b_ref, o_ref, y0_s, y1_s, acc_ref):
    i = pl.program_id(0)
    k = pl.program_id(1)
    nk = pl.num_programs(1)
    kw = a0_ref.shape[1]

    @pl.when(i == 0)
    def _compute_y():
        xk = x_k_ref[...]
        y0_s[k] = jnp.dot(xk, w0_ref[...], preferred_element_type=jnp.float32)
        y1_s[k] = jnp.dot(xk, w1_ref[...], preferred_element_type=jnp.float32)

    @pl.when(k == 0)
    def _init_acc():
        acc_ref[...] = jnp.dot(x_i_ref[...], ws_ref[...],
                               preferred_element_type=jnp.float32) + b_ref[...]

    acc_ref[...] += (
        jnp.dot(a0_ref[...], y0_s[k, pl.ds(0, kw), :],
                preferred_element_type=jnp.float32)
        + jnp.dot(a0b_ref[...], y0_s[k, pl.ds(kw, kw), :],
                  preferred_element_type=jnp.float32)
        + jnp.dot(a1_ref[...], y1_s[k, pl.ds(0, kw), :],
                  preferred_element_type=jnp.float32)
        + jnp.dot(a1b_ref[...], y1_s[k, pl.ds(kw, kw), :],
                  preferred_element_type=jnp.float32))

    @pl.when(k == nk - 1)
    def _epilogue():
        o_ref[...] = jnp.maximum(acc_ref[...], 0.0)


@functools.partial(jax.jit, static_argnames=("tm", "tk"))
def _rgcn(x, a0, a1, ws, w0, w1, b, tm=256, tk=8192):
    n, f = x.shape
    u = ws.shape[1]
    ni = n // tm
    nk = n // tk
    b2 = b.reshape(1, u)
    out = pl.pallas_call(
        _rgcn_body,
        grid=(ni, nk),
        in_specs=[
            pl.BlockSpec((tk, f), lambda i, k: (jnp.where(i == 0, k, 0), 0)),
            pl.BlockSpec((tm, f), lambda i, k: (i, 0)),
            pl.BlockSpec((tm, tk // 2), lambda i, k: (i, 2 * k)),
            pl.BlockSpec((tm, tk // 2), lambda i, k: (i, 2 * k + 1)),
            pl.BlockSpec((tm, tk // 2), lambda i, k: (i, 2 * k)),
            pl.BlockSpec((tm, tk // 2), lambda i, k: (i, 2 * k + 1)),
            pl.BlockSpec((f, u), lambda i, k: (0, 0)),
            pl.BlockSpec((f, u), lambda i, k: (0, 0)),
            pl.BlockSpec((f, u), lambda i, k: (0, 0)),
            pl.BlockSpec((1, u), lambda i, k: (0, 0)),
        ],
        out_specs=pl.BlockSpec((tm, u), lambda i, k: (i, 0)),
        out_shape=jax.ShapeDtypeStruct((n, u), jnp.float32),
        scratch_shapes=[
            pltpu.VMEM((nk, tk, u), jnp.float32),
            pltpu.VMEM((nk, tk, u), jnp.float32),
            pltpu.VMEM((tm, u), jnp.float32),
        ],
        compiler_params=pltpu.CompilerParams(
            dimension_semantics=("arbitrary", "arbitrary")),
    )(x, x, a0, a0, a1, a1, ws, w0, w1, b2)
    return out


def kernel(features, A_0, A_1, self_kernel, rel_kernel_0, rel_kernel_1, bias):
    x = features[0]
    out = _rgcn(x, A_0, A_1, self_kernel, rel_kernel_0, rel_kernel_1, bias)
    return out[None, ...]
